# baseline (speedup 1.0000x reference)
import functools

import jax
import jax.numpy as jnp
from jax import lax
from jax.experimental import pallas as pl
from jax.experimental.pallas import tpu as pltpu
from jax.experimental.pallas import tpu_sc as plsc

NC = 2
NS = 16
L = 16
NW = NC * NS
B = 16384
BPW = B // NW
CHUNK = 128
NCHUNK = BPW // CHUNK
D = 16
G = BPW // L

_mesh = plsc.VectorSubcoreMesh(core_axis_name="c", subcore_axis_name="s")


@functools.partial(
    pl.kernel,
    out_type=jax.ShapeDtypeStruct((B,), jnp.float32),
    mesh=_mesh,
    compiler_params=pltpu.CompilerParams(
        needs_layout_passes=False,
        use_tc_tiling_on_sc=False,
    ),
    scratch_types=[
        pltpu.VMEM((BPW,), jnp.int32),
        pltpu.VMEM((BPW,), jnp.int32),
        pltpu.VMEM((BPW, D), jnp.float32),
        pltpu.VMEM((BPW, D), jnp.float32),
        pltpu.VMEM((BPW * D,), jnp.float32),
        pltpu.VMEM((BPW * D,), jnp.float32),
        pltpu.VMEM((BPW,), jnp.float32),
        pltpu.VMEM((BPW,), jnp.float32),
        pltpu.SemaphoreType.DMA,
    ],
)
def _mirt_sc(uid_hbm, qid_hbm, theta_hbm, a_hbm, b_hbm, out_hbm,
             uid_v, qid_v, th_v, a_v, th_t, a_t, b_v, out_v, sem):
    wid = lax.axis_index("s") * NC + lax.axis_index("c")
    base = wid * BPW
    pltpu.sync_copy(uid_hbm.at[pl.ds(base, BPW)], uid_v)
    pltpu.sync_copy(qid_hbm.at[pl.ds(base, BPW)], qid_v)

    copies = []
    for j in range(NCHUNK):
        sl = pl.ds(j * CHUNK, CHUNK)
        copies.append(pltpu.make_async_copy(theta_hbm.at[uid_v.at[sl]], th_v.at[sl], sem))
        copies.append(pltpu.make_async_copy(a_hbm.at[qid_v.at[sl]], a_v.at[sl], sem))
        copies.append(pltpu.make_async_copy(b_hbm.at[qid_v.at[sl]], b_v.at[sl], sem))
    for c in copies:
        c.start()
    for c in copies:
        c.wait()

    lanes = lax.iota(jnp.int32, L)
    col_base = lanes * BPW

    def transpose_row(r, carry):
        idx = col_base + r
        plsc.store_scatter(th_t, [idx], th_v[r, :])
        plsc.store_scatter(a_t, [idx], a_v[r, :])
        return carry

    lax.fori_loop(0, BPW, transpose_row, 0)

    def group(g, carry):
        row0 = pl.multiple_of(g * L, L)
        acc = jnp.zeros((L,), jnp.float32)
        for c in range(D):
            av = a_t[pl.ds(c * BPW + row0, L)]
            tv = th_t[pl.ds(c * BPW + row0, L)]
            acc = acc + tv / (1.0 + jnp.exp(-av))
        bb = b_v[pl.ds(row0, L)]
        out_v[pl.ds(row0, L)] = 1.0 / (1.0 + jnp.exp(bb - acc))
        return carry

    lax.fori_loop(0, G, group, 0)
    pltpu.sync_copy(out_v, out_hbm.at[pl.ds(base, BPW)])


def kernel(user_id, question_id, theta_w, a_w, b_w):
    return _mirt_sc(user_id.astype(jnp.int32), question_id.astype(jnp.int32),
                    theta_w, a_w, jnp.reshape(b_w, (-1,)))
